# baseline (speedup 1.0000x reference)
import functools

import jax
import jax.numpy as jnp
from jax import lax
from jax.experimental import pallas as pl
from jax.experimental.pallas import tpu as pltpu
from jax.experimental.pallas import tpu_sc as plsc

_B = 4096
_SEQ = 64
_N_EMB = 26
_TOTAL_INPUT = 52
_VOCAB = 100000
_DIM = 64
_LANES = 16
_COLS = _DIM // _LANES

_NC = 2
_NS = 16
_NW = _NC * _NS
_BPW = _B // _NW
_IDX_PW = _BPW * _N_EMB
_CB = 4
_CIDX = _CB * _N_EMB
_NCHUNK = _BPW // _CB

_BV = 14336
_VOCAB_PAD = 100352
_NBV = _VOCAB_PAD // _BV


def _transpose_body(a_ref, b_ref, out_ref):
    out_ref[:, :_DIM] = a_ref[0].T
    out_ref[:, _DIM:] = b_ref[0].T


@functools.partial(jax.jit, static_argnums=(1, 2))
def _transpose_pairs(tables_t, first_pair, npairs):
    return pl.pallas_call(
        _transpose_body,
        out_shape=jax.ShapeDtypeStruct((npairs * _VOCAB_PAD, 2 * _DIM),
                                       jnp.float32),
        grid=(npairs, _NBV),
        in_specs=[
            pl.BlockSpec((1, _DIM, _BV),
                         lambda k, c: (2 * (k + first_pair), 0, c)),
            pl.BlockSpec((1, _DIM, _BV),
                         lambda k, c: (2 * (k + first_pair) + 1, 0, c)),
        ],
        out_specs=pl.BlockSpec((_BV, 2 * _DIM),
                               lambda k, c: (k * _NBV + c, 0)),
        compiler_params=pltpu.CompilerParams(
            vmem_limit_bytes=100 * 1024 * 1024),
    )(tables_t, tables_t)


def _make_emb_body(n_emb):
    idx_pw = _BPW * n_emb
    cidx = _CB * n_emb

    def _emb_body(tab_hbm, idx_hbm, out_hbm, idx_v, half_v, rows_v, out_v,
                  sem0, sem1):
        wid = lax.axis_index("s") * _NC + lax.axis_index("c")
        base = wid * idx_pw

        pltpu.sync_copy(idx_hbm.at[pl.ds(base, idx_pw)], idx_v)

        def xform(p, carry):
            s = pl.multiple_of(p * _LANES, _LANES)
            pos = lax.iota(jnp.int32, _LANES) + s
            j = lax.rem(pos, n_emb)
            half_v[pl.ds(s, _LANES)] = lax.bitwise_and(j, 1) * _DIM
            idx_v[pl.ds(s, _LANES)] = (
                idx_v[pl.ds(s, _LANES)]
                + lax.shift_right_logical(j, 1) * _VOCAB_PAD
            )
            return carry

        lax.fori_loop(0, idx_pw // _LANES, xform, 0)

        def gather(c, buf, sem):
            return pltpu.async_copy(
                tab_hbm.at[idx_v.at[pl.ds(c * cidx, cidx)]],
                rows_v.at[buf],
                sem,
            )

        def wait(c, buf, sem):
            pltpu.make_async_copy(
                tab_hbm.at[idx_v.at[pl.ds(c * cidx, cidx)]],
                rows_v.at[buf],
                sem,
            ).wait()

        def accum(c, buf):
            for bi in range(_CB):
                def jbody(j, accs, _bi=bi):
                    r = _bi * n_emb + j
                    off = half_v[pl.ds(c * cidx + r, _LANES)][0]
                    return tuple(
                        accs[t]
                        + rows_v[buf, r, pl.ds(off + t * _LANES, _LANES)]
                        for t in range(_COLS)
                    )

                accs = lax.fori_loop(
                    0, n_emb, jbody,
                    tuple(jnp.zeros((_LANES,), jnp.float32)
                          for _ in range(_COLS)),
                )
                prow = c * (_CB // 2) + bi // 2
                lane = (bi % 2) * _DIM
                for t in range(_COLS):
                    out_v[prow, pl.ds(lane + t * _LANES, _LANES)] = accs[t]

        gather(0, 0, sem0)
        gather(1, 1, sem1)

        def outer(i, carry):
            c0 = 2 * i
            wait(c0, 0, sem0)
            accum(c0, 0)

            @pl.when(c0 + 2 < _NCHUNK)
            def _():
                gather(c0 + 2, 0, sem0)

            c1 = c0 + 1
            wait(c1, 1, sem1)
            accum(c1, 1)

            @pl.when(c1 + 2 < _NCHUNK)
            def _():
                gather(c1 + 2, 1, sem1)

            return carry

        lax.fori_loop(0, _NCHUNK // 2, outer, 0)

        pltpu.sync_copy(out_v,
                        out_hbm.at[pl.ds(wid * (_BPW // 2), _BPW // 2)])

    return _emb_body


@functools.partial(jax.jit, static_argnums=2)
def _embedding_sum(tab_pairs, flat_idx, n_emb):
    idx_pw = _BPW * n_emb
    cidx = _CB * n_emb
    mesh = plsc.VectorSubcoreMesh(core_axis_name="c", subcore_axis_name="s")
    return pl.kernel(
        _make_emb_body(n_emb),
        out_type=jax.ShapeDtypeStruct((_B // 2, 2 * _DIM), jnp.float32),
        mesh=mesh,
        scratch_types=[
            pltpu.VMEM((idx_pw,), jnp.int32),
            pltpu.VMEM((idx_pw + _LANES,), jnp.int32),
            pltpu.VMEM((2, cidx, 2 * _DIM), jnp.float32),
            pltpu.VMEM((_BPW // 2, 2 * _DIM), jnp.float32),
            pltpu.SemaphoreType.DMA,
            pltpu.SemaphoreType.DMA,
        ],
    )(tab_pairs, flat_idx)


def kernel(input, mask_key, tables):
    numeric = input[:, :, _N_EMB:].astype(jnp.float32)
    past_seq = numeric[:, :_TOTAL_INPUT, :]
    future_seq = numeric[:, _TOTAL_INPUT:, :]
    tables_t = jnp.swapaxes(tables, 1, 2)
    tab_a = _transpose_pairs(tables_t, 0, 7)
    tab_b = _transpose_pairs(tables_t, 7, 6)
    idx_a = input[:, 0, :14].reshape(-1)
    idx_b = input[:, 0, 14:_N_EMB].reshape(-1)
    emb_a = _embedding_sum(tab_a, idx_a, 14)
    emb_b = _embedding_sum(tab_b, idx_b, 12)
    embedded_output = (emb_a + emb_b).reshape(_B, _DIM)
    return (past_seq, future_seq, embedded_output)


# speedup vs baseline: 1.0102x; 1.0102x over previous
import functools

import jax
import jax.numpy as jnp
from jax import lax
from jax.experimental import pallas as pl
from jax.experimental.pallas import tpu as pltpu
from jax.experimental.pallas import tpu_sc as plsc

_B = 4096
_SEQ = 64
_N_EMB = 26
_TOTAL_INPUT = 52
_VOCAB = 100000
_DIM = 64
_LANES = 16
_COLS = _DIM // _LANES

_NC = 2
_NS = 16
_NW = _NC * _NS
_BPW = _B // _NW
_IDX_PW = _BPW * _N_EMB
_CB = 8
_NCHUNK = _BPW // _CB

_BV = 14336
_VOCAB_PAD = 100352
_NBV = _VOCAB_PAD // _BV


def _transpose_body(a_ref, b_ref, out_ref):
    out_ref[:, :_DIM] = a_ref[0].T
    out_ref[:, _DIM:] = b_ref[0].T


@functools.partial(jax.jit, static_argnums=(1, 2))
def _transpose_pairs(tables_t, first_pair, npairs):
    return pl.pallas_call(
        _transpose_body,
        out_shape=jax.ShapeDtypeStruct((npairs * _VOCAB_PAD, 2 * _DIM),
                                       jnp.float32),
        grid=(npairs, _NBV),
        in_specs=[
            pl.BlockSpec((1, _DIM, _BV),
                         lambda k, c: (2 * (k + first_pair), 0, c)),
            pl.BlockSpec((1, _DIM, _BV),
                         lambda k, c: (2 * (k + first_pair) + 1, 0, c)),
        ],
        out_specs=pl.BlockSpec((_BV, 2 * _DIM),
                               lambda k, c: (k * _NBV + c, 0)),
        compiler_params=pltpu.CompilerParams(
            vmem_limit_bytes=100 * 1024 * 1024),
    )(tables_t, tables_t)


def _make_emb_body(n_emb):
    idx_pw = _BPW * n_emb
    cidx = _CB * n_emb

    def _emb_body(tab_hbm, idx_hbm, out_hbm, idx_v, half_v, rows_v, out_v,
                  sem0, sem1):
        wid = lax.axis_index("s") * _NC + lax.axis_index("c")
        base = wid * idx_pw

        pltpu.sync_copy(idx_hbm.at[pl.ds(base, idx_pw)], idx_v)

        def xform(p, carry):
            s = pl.multiple_of(p * _LANES, _LANES)
            pos = lax.iota(jnp.int32, _LANES) + s
            j = lax.rem(pos, n_emb)
            half_v[pl.ds(s, _LANES)] = lax.bitwise_and(j, 1) * _DIM
            idx_v[pl.ds(s, _LANES)] = (
                idx_v[pl.ds(s, _LANES)]
                + lax.shift_right_logical(j, 1) * _VOCAB_PAD
            )
            return carry

        lax.fori_loop(0, idx_pw // _LANES, xform, 0)

        def gather(c, buf, sem):
            return pltpu.async_copy(
                tab_hbm.at[idx_v.at[pl.ds(c * cidx, cidx)]],
                rows_v.at[buf],
                sem,
            )

        def wait(c, buf, sem):
            pltpu.make_async_copy(
                tab_hbm.at[idx_v.at[pl.ds(c * cidx, cidx)]],
                rows_v.at[buf],
                sem,
            ).wait()

        def accum(c, buf):
            for bi in range(_CB):
                def jbody(j, accs, _bi=bi):
                    r = _bi * n_emb + j
                    off = half_v[pl.ds(c * cidx + r, _LANES)][0]
                    return tuple(
                        accs[t]
                        + rows_v[buf, r, pl.ds(off + t * _LANES, _LANES)]
                        for t in range(_COLS)
                    )

                accs = lax.fori_loop(
                    0, n_emb, jbody,
                    tuple(jnp.zeros((_LANES,), jnp.float32)
                          for _ in range(_COLS)),
                )
                prow = c * (_CB // 2) + bi // 2
                lane = (bi % 2) * _DIM
                for t in range(_COLS):
                    out_v[prow, pl.ds(lane + t * _LANES, _LANES)] = accs[t]

        gather(0, 0, sem0)
        gather(1, 1, sem1)

        def outer(i, carry):
            c0 = 2 * i
            wait(c0, 0, sem0)
            accum(c0, 0)

            @pl.when(c0 + 2 < _NCHUNK)
            def _():
                gather(c0 + 2, 0, sem0)

            c1 = c0 + 1
            wait(c1, 1, sem1)
            accum(c1, 1)

            @pl.when(c1 + 2 < _NCHUNK)
            def _():
                gather(c1 + 2, 1, sem1)

            return carry

        lax.fori_loop(0, _NCHUNK // 2, outer, 0)

        pltpu.sync_copy(out_v,
                        out_hbm.at[pl.ds(wid * (_BPW // 2), _BPW // 2)])

    return _emb_body


@functools.partial(jax.jit, static_argnums=2)
def _embedding_sum(tab_pairs, flat_idx, n_emb):
    idx_pw = _BPW * n_emb
    cidx = _CB * n_emb
    mesh = plsc.VectorSubcoreMesh(core_axis_name="c", subcore_axis_name="s")
    return pl.kernel(
        _make_emb_body(n_emb),
        out_type=jax.ShapeDtypeStruct((_B // 2, 2 * _DIM), jnp.float32),
        mesh=mesh,
        scratch_types=[
            pltpu.VMEM((idx_pw,), jnp.int32),
            pltpu.VMEM((idx_pw + _LANES,), jnp.int32),
            pltpu.VMEM((2, cidx, 2 * _DIM), jnp.float32),
            pltpu.VMEM((_BPW // 2, 2 * _DIM), jnp.float32),
            pltpu.SemaphoreType.DMA,
            pltpu.SemaphoreType.DMA,
        ],
    )(tab_pairs, flat_idx)


def kernel(input, mask_key, tables):
    numeric = input[:, :, _N_EMB:].astype(jnp.float32)
    past_seq = numeric[:, :_TOTAL_INPUT, :]
    future_seq = numeric[:, _TOTAL_INPUT:, :]
    tables_t = jnp.swapaxes(tables, 1, 2)
    tab_a = _transpose_pairs(tables_t, 0, 8)
    tab_b = _transpose_pairs(tables_t, 8, 5)
    idx_a = input[:, 0, :16].reshape(-1)
    idx_b = input[:, 0, 16:_N_EMB].reshape(-1)
    emb_a = _embedding_sum(tab_a, idx_a, 16)
    emb_b = _embedding_sum(tab_b, idx_b, 10)
    embedded_output = (emb_a + emb_b).reshape(_B, _DIM)
    return (past_seq, future_seq, embedded_output)


# speedup vs baseline: 1.0121x; 1.0018x over previous
import functools

import jax
import jax.numpy as jnp
from jax import lax
from jax.experimental import pallas as pl
from jax.experimental.pallas import tpu as pltpu
from jax.experimental.pallas import tpu_sc as plsc

_B = 4096
_SEQ = 64
_N_EMB = 26
_TOTAL_INPUT = 52
_VOCAB = 100000
_DIM = 64
_LANES = 16
_COLS = _DIM // _LANES

_NC = 2
_NS = 16
_NW = _NC * _NS
_BPW = _B // _NW
_IDX_PW = _BPW * _N_EMB
_CB = 8
_NCHUNK = _BPW // _CB

_BV = 25088
_VOCAB_PAD = 100352
_NBV = _VOCAB_PAD // _BV


def _transpose_body(a_ref, b_ref, out_ref):
    out_ref[:, :_DIM] = a_ref[0].T
    out_ref[:, _DIM:] = b_ref[0].T


@functools.partial(jax.jit, static_argnums=(1, 2))
def _transpose_pairs(tables_t, first_pair, npairs):
    return pl.pallas_call(
        _transpose_body,
        out_shape=jax.ShapeDtypeStruct((npairs * _VOCAB_PAD, 2 * _DIM),
                                       jnp.float32),
        grid=(npairs, _NBV),
        in_specs=[
            pl.BlockSpec((1, _DIM, _BV),
                         lambda k, c: (2 * (k + first_pair), 0, c)),
            pl.BlockSpec((1, _DIM, _BV),
                         lambda k, c: (2 * (k + first_pair) + 1, 0, c)),
        ],
        out_specs=pl.BlockSpec((_BV, 2 * _DIM),
                               lambda k, c: (k * _NBV + c, 0)),
        compiler_params=pltpu.CompilerParams(
            vmem_limit_bytes=100 * 1024 * 1024),
    )(tables_t, tables_t)


def _make_emb_body(n_emb):
    idx_pw = _BPW * n_emb
    cidx = _CB * n_emb

    def _emb_body(tab_hbm, idx_hbm, out_hbm, idx_v, half_v, rows_v, out_v,
                  sem0, sem1):
        wid = lax.axis_index("s") * _NC + lax.axis_index("c")
        base = wid * idx_pw

        pltpu.sync_copy(idx_hbm.at[pl.ds(base, idx_pw)], idx_v)

        def xform(p, carry):
            s = pl.multiple_of(p * _LANES, _LANES)
            pos = lax.iota(jnp.int32, _LANES) + s
            j = lax.rem(pos, n_emb)
            half_v[pl.ds(s, _LANES)] = lax.bitwise_and(j, 1) * _DIM
            idx_v[pl.ds(s, _LANES)] = (
                idx_v[pl.ds(s, _LANES)]
                + lax.shift_right_logical(j, 1) * _VOCAB_PAD
            )
            return carry

        lax.fori_loop(0, idx_pw // _LANES, xform, 0)

        def gather(c, buf, sem):
            return pltpu.async_copy(
                tab_hbm.at[idx_v.at[pl.ds(c * cidx, cidx)]],
                rows_v.at[buf],
                sem,
            )

        def wait(c, buf, sem):
            pltpu.make_async_copy(
                tab_hbm.at[idx_v.at[pl.ds(c * cidx, cidx)]],
                rows_v.at[buf],
                sem,
            ).wait()

        def accum(c, buf):
            for bi in range(_CB):
                def jbody(j, accs, _bi=bi):
                    r = _bi * n_emb + j
                    off = half_v[pl.ds(c * cidx + r, _LANES)][0]
                    return tuple(
                        accs[t]
                        + rows_v[buf, r, pl.ds(off + t * _LANES, _LANES)]
                        for t in range(_COLS)
                    )

                accs = lax.fori_loop(
                    0, n_emb, jbody,
                    tuple(jnp.zeros((_LANES,), jnp.float32)
                          for _ in range(_COLS)),
                )
                prow = c * (_CB // 2) + bi // 2
                lane = (bi % 2) * _DIM
                for t in range(_COLS):
                    out_v[prow, pl.ds(lane + t * _LANES, _LANES)] = accs[t]

        gather(0, 0, sem0)
        gather(1, 1, sem1)

        def outer(i, carry):
            c0 = 2 * i
            wait(c0, 0, sem0)
            accum(c0, 0)

            @pl.when(c0 + 2 < _NCHUNK)
            def _():
                gather(c0 + 2, 0, sem0)

            c1 = c0 + 1
            wait(c1, 1, sem1)
            accum(c1, 1)

            @pl.when(c1 + 2 < _NCHUNK)
            def _():
                gather(c1 + 2, 1, sem1)

            return carry

        lax.fori_loop(0, _NCHUNK // 2, outer, 0)

        pltpu.sync_copy(out_v,
                        out_hbm.at[pl.ds(wid * (_BPW // 2), _BPW // 2)])

    return _emb_body


@functools.partial(jax.jit, static_argnums=2)
def _embedding_sum(tab_pairs, flat_idx, n_emb):
    idx_pw = _BPW * n_emb
    cidx = _CB * n_emb
    mesh = plsc.VectorSubcoreMesh(core_axis_name="c", subcore_axis_name="s")
    return pl.kernel(
        _make_emb_body(n_emb),
        out_type=jax.ShapeDtypeStruct((_B // 2, 2 * _DIM), jnp.float32),
        mesh=mesh,
        scratch_types=[
            pltpu.VMEM((idx_pw,), jnp.int32),
            pltpu.VMEM((idx_pw + _LANES,), jnp.int32),
            pltpu.VMEM((2, cidx, 2 * _DIM), jnp.float32),
            pltpu.VMEM((_BPW // 2, 2 * _DIM), jnp.float32),
            pltpu.SemaphoreType.DMA,
            pltpu.SemaphoreType.DMA,
        ],
    )(tab_pairs, flat_idx)


def kernel(input, mask_key, tables):
    numeric = input[:, :, _N_EMB:].astype(jnp.float32)
    past_seq = numeric[:, :_TOTAL_INPUT, :]
    future_seq = numeric[:, _TOTAL_INPUT:, :]
    tables_t = jnp.swapaxes(tables, 1, 2)
    tab_a = _transpose_pairs(tables_t, 0, 8)
    tab_b = _transpose_pairs(tables_t, 8, 5)
    idx_a = input[:, 0, :16].reshape(-1)
    idx_b = input[:, 0, 16:_N_EMB].reshape(-1)
    emb_a = _embedding_sum(tab_a, idx_a, 16)
    emb_b = _embedding_sum(tab_b, idx_b, 10)
    embedded_output = (emb_a + emb_b).reshape(_B, _DIM)
    return (past_seq, future_seq, embedded_output)
